# baseline (speedup 1.0000x reference)
import jax
import jax.numpy as jnp
from jax.experimental import pallas as pl
from jax.experimental.pallas import tpu as pltpu

N = 16384
C = 10
CHUNKS = 2
CW = N // CHUNKS


def _onehot_block(lbl_ref, out_hbm, stage, sem):
    classes = jax.lax.broadcasted_iota(jnp.int32, (C, CW), 0)
    for k in range(CHUNKS):
        lbl = lbl_ref[pl.ds(k * CW, CW)]
        stage[:, pl.ds(k * CW, CW)] = jnp.where(
            lbl[None, :] == classes, 1.0, 0.0
        ).astype(jnp.float32)
        pltpu.make_async_copy(
            stage.at[:, pl.ds(k * CW, CW)],
            out_hbm.at[:, pl.ds(k * CW, CW)],
            sem,
        ).start()
    for k in range(CHUNKS):
        pltpu.make_async_copy(
            stage.at[:, pl.ds(k * CW, CW)],
            out_hbm.at[:, pl.ds(k * CW, CW)],
            sem,
        ).wait()


_onehot_tc = pl.pallas_call(
    _onehot_block,
    out_specs=pl.BlockSpec(memory_space=pltpu.MemorySpace.HBM),
    out_shape=jax.ShapeDtypeStruct((C, N), jnp.float32),
    scratch_shapes=[
        pltpu.VMEM((C, N), jnp.float32),
        pltpu.SemaphoreType.DMA,
    ],
)


@jax.jit
def kernel(label):
    return _onehot_tc(label).T


# speedup vs baseline: 1.0030x; 1.0030x over previous
import jax
import jax.numpy as jnp
from jax.experimental import pallas as pl
from jax.experimental.pallas import tpu as pltpu

N = 16384
C = 10
CHUNK_SIZES = (8192, 4096, 2048, 2048)


def _onehot_block(lbl_hbm, out_hbm, lblv, stage, lsem, sem):
    pltpu.make_async_copy(lbl_hbm, lblv, lsem).start()
    pltpu.make_async_copy(lbl_hbm, lblv, lsem).wait()
    off = 0
    for cw in CHUNK_SIZES:
        lbl = lblv[pl.ds(off, cw)]
        classes = jax.lax.broadcasted_iota(jnp.int32, (C, cw), 0)
        stage[:, pl.ds(off, cw)] = jnp.where(
            lbl[None, :] == classes, 1.0, 0.0
        ).astype(jnp.float32)
        pltpu.make_async_copy(
            stage.at[:, pl.ds(off, cw)],
            out_hbm.at[:, pl.ds(off, cw)],
            sem,
        ).start()
        off += cw
    pltpu.make_async_copy(stage, out_hbm, sem).wait()


_onehot_tc = pl.pallas_call(
    _onehot_block,
    in_specs=[pl.BlockSpec(memory_space=pltpu.MemorySpace.HBM)],
    out_specs=pl.BlockSpec(memory_space=pltpu.MemorySpace.HBM),
    out_shape=jax.ShapeDtypeStruct((C, N), jnp.float32),
    scratch_shapes=[
        pltpu.VMEM((N,), jnp.int32),
        pltpu.VMEM((C, N), jnp.float32),
        pltpu.SemaphoreType.DMA,
        pltpu.SemaphoreType.DMA,
    ],
)


@jax.jit
def kernel(label):
    return _onehot_tc(label).T


# speedup vs baseline: 1.0283x; 1.0252x over previous
import jax
import jax.numpy as jnp
from jax.experimental import pallas as pl
from jax.experimental.pallas import tpu as pltpu

N = 16384
C = 10
CHUNK_SIZES = (2048, 4096, 5120, 5120)


def _onehot_block(lbl_ref, out_hbm, stage, sem):
    off = 0
    for cw in CHUNK_SIZES:
        lbl = lbl_ref[pl.ds(off, cw)]
        classes = jax.lax.broadcasted_iota(jnp.int32, (C, cw), 0)
        stage[:, pl.ds(off, cw)] = jnp.where(
            lbl[None, :] == classes, 1.0, 0.0
        ).astype(jnp.float32)
        pltpu.make_async_copy(
            stage.at[:, pl.ds(off, cw)],
            out_hbm.at[:, pl.ds(off, cw)],
            sem,
        ).start()
        off += cw
    pltpu.make_async_copy(stage, out_hbm, sem).wait()


_onehot_tc = pl.pallas_call(
    _onehot_block,
    out_specs=pl.BlockSpec(memory_space=pltpu.MemorySpace.HBM),
    out_shape=jax.ShapeDtypeStruct((C, N), jnp.float32),
    scratch_shapes=[
        pltpu.VMEM((C, N), jnp.float32),
        pltpu.SemaphoreType.DMA,
    ],
)


@jax.jit
def kernel(label):
    return _onehot_tc(label).T
